# Initial kernel scaffold; baseline (speedup 1.0000x reference)
import jax
import jax.numpy as jnp
from jax.experimental import pallas as pl


def kernel(xi, xj):
    raise NotImplementedError("write your pallas kernel here")


# speedup vs baseline: 8.2580x; 8.2580x over previous
import functools

import jax
import jax.numpy as jnp
from jax.experimental import pallas as pl
from jax.experimental.pallas import tpu as pltpu

_BM = 512
_BN = 512
_NEG = -1e30


def _koleo_body(nrows, ncols, eps, xi_ref, xj_ref, out_ref, max_ref):
    i = pl.program_id(0)
    j = pl.program_id(1)

    xi_blk = xi_ref[...]
    xj_blk = xj_ref[...]

    b2 = jnp.sum(xj_blk * xj_blk, axis=1, keepdims=True)
    s = jax.lax.dot_general(
        xi_blk, xj_blk, (((1,), (1,)), ((), ())),
        preferred_element_type=jnp.float32)
    score = b2.T - 2.0 * s

    rows = i * _BM + jax.lax.broadcasted_iota(jnp.int32, (_BM, _BN), 0)
    cols = j * _BN + jax.lax.broadcasted_iota(jnp.int32, (_BM, _BN), 1)
    score = jnp.where(rows == cols, _NEG, score)

    m = jnp.max(score, axis=1, keepdims=True)

    @pl.when(j == 0)
    def _():
        max_ref[...] = m

    @pl.when(j > 0)
    def _():
        max_ref[...] = jnp.maximum(max_ref[...], m)

    @pl.when((i == 0) & (j == 0))
    def _():
        out_ref[...] = jnp.zeros((1, 1), jnp.float32)

    @pl.when(j == ncols - 1)
    def _():
        a2 = jnp.sum(xi_blk * xi_blk, axis=1, keepdims=True)
        d2 = a2 + max_ref[...]
        lg = jnp.log(1.0 / (d2 * 0.5 + 1.0) ** 2 + eps)
        out_ref[...] += jnp.sum(lg, keepdims=True)


def kernel(xi, xj):
    eps = 1e-08
    n, k = xi.shape
    nrows = n // _BM
    ncols = n // _BN

    out = pl.pallas_call(
        functools.partial(_koleo_body, nrows, ncols, eps),
        grid=(nrows, ncols),
        in_specs=[
            pl.BlockSpec((_BM, k), lambda i, j: (i, 0)),
            pl.BlockSpec((_BN, k), lambda i, j: (j, 0)),
        ],
        out_specs=pl.BlockSpec((1, 1), lambda i, j: (0, 0)),
        out_shape=jax.ShapeDtypeStruct((1, 1), jnp.float32),
        scratch_shapes=[pltpu.VMEM((_BM, 1), jnp.float32)],
        compiler_params=pltpu.CompilerParams(
            dimension_semantics=("arbitrary", "arbitrary")),
    )(xi, xj)
    return out[0, 0] / n


# speedup vs baseline: 8.3282x; 1.0085x over previous
import functools

import jax
import jax.numpy as jnp
from jax.experimental import pallas as pl
from jax.experimental.pallas import tpu as pltpu

_BM = 512
_BN = 512
_NEG = -1e30


def _koleo_body(nrows, ncols, eps, xi_ref, xj_ref, out_ref, max_ref):
    i = pl.program_id(0)
    j = pl.program_id(1)

    xi_blk = xi_ref[...]
    xj_blk = xj_ref[...]

    b2 = jnp.sum(xj_blk * xj_blk, axis=1, keepdims=True)
    s = jax.lax.dot_general(
        xi_blk.astype(jnp.bfloat16), xj_blk.astype(jnp.bfloat16),
        (((1,), (1,)), ((), ())),
        preferred_element_type=jnp.float32)
    score = b2.T - 2.0 * s

    rows = i * _BM + jax.lax.broadcasted_iota(jnp.int32, (_BM, _BN), 0)
    cols = j * _BN + jax.lax.broadcasted_iota(jnp.int32, (_BM, _BN), 1)
    score = jnp.where(rows == cols, _NEG, score)

    m = jnp.max(score, axis=1, keepdims=True)

    @pl.when(j == 0)
    def _():
        max_ref[...] = m

    @pl.when(j > 0)
    def _():
        max_ref[...] = jnp.maximum(max_ref[...], m)

    @pl.when((i == 0) & (j == 0))
    def _():
        out_ref[...] = jnp.zeros((1, 1), jnp.float32)

    @pl.when(j == ncols - 1)
    def _():
        a2 = jnp.sum(xi_blk * xi_blk, axis=1, keepdims=True)
        d2 = a2 + max_ref[...]
        lg = jnp.log(1.0 / (d2 * 0.5 + 1.0) ** 2 + eps)
        out_ref[...] += jnp.sum(lg, keepdims=True)


def kernel(xi, xj):
    eps = 1e-08
    n, k = xi.shape
    nrows = n // _BM
    ncols = n // _BN

    out = pl.pallas_call(
        functools.partial(_koleo_body, nrows, ncols, eps),
        grid=(nrows, ncols),
        in_specs=[
            pl.BlockSpec((_BM, k), lambda i, j: (i, 0)),
            pl.BlockSpec((_BN, k), lambda i, j: (j, 0)),
        ],
        out_specs=pl.BlockSpec((1, 1), lambda i, j: (0, 0)),
        out_shape=jax.ShapeDtypeStruct((1, 1), jnp.float32),
        scratch_shapes=[pltpu.VMEM((_BM, 1), jnp.float32)],
        compiler_params=pltpu.CompilerParams(
            dimension_semantics=("arbitrary", "arbitrary")),
    )(xi, xj)
    return out[0, 0] / n


# speedup vs baseline: 13.4799x; 1.6186x over previous
import functools

import jax
import jax.numpy as jnp
from jax.experimental import pallas as pl
from jax.experimental.pallas import tpu as pltpu

_BM = 512
_NEG = -1e30


def _koleo_body(n, eps, xi_ref, xj_ref, out_ref):
    i = pl.program_id(0)

    xi_blk = xi_ref[...]
    xj_all = xj_ref[...]

    b2 = jnp.sum(xj_all * xj_all, axis=1, keepdims=True)
    s = jax.lax.dot_general(
        xi_blk.astype(jnp.bfloat16), xj_all.astype(jnp.bfloat16),
        (((1,), (1,)), ((), ())),
        preferred_element_type=jnp.float32)
    score = b2.T - 2.0 * s

    rows = i * _BM + jax.lax.broadcasted_iota(jnp.int32, (_BM, n), 0)
    cols = jax.lax.broadcasted_iota(jnp.int32, (_BM, n), 1)
    score = jnp.where(rows == cols, _NEG, score)

    m = jnp.max(score, axis=1, keepdims=True)
    a2 = jnp.sum(xi_blk * xi_blk, axis=1, keepdims=True)
    d2 = a2 + m
    lg = jnp.log(1.0 / (d2 * 0.5 + 1.0) ** 2 + eps)

    @pl.when(i == 0)
    def _():
        out_ref[...] = jnp.zeros((1, 1), jnp.float32)

    out_ref[...] += jnp.sum(lg, keepdims=True)


def kernel(xi, xj):
    eps = 1e-08
    n, k = xi.shape

    out = pl.pallas_call(
        functools.partial(_koleo_body, n, eps),
        grid=(n // _BM,),
        in_specs=[
            pl.BlockSpec((_BM, k), lambda i: (i, 0)),
            pl.BlockSpec((n, k), lambda i: (0, 0)),
        ],
        out_specs=pl.BlockSpec((1, 1), lambda i: (0, 0)),
        out_shape=jax.ShapeDtypeStruct((1, 1), jnp.float32),
        compiler_params=pltpu.CompilerParams(
            dimension_semantics=("arbitrary",)),
    )(xi, xj)
    return out[0, 0] / n


# speedup vs baseline: 14.6949x; 1.0901x over previous
import functools

import jax
import jax.numpy as jnp
from jax.experimental import pallas as pl
from jax.experimental.pallas import tpu as pltpu

_BM = 512
_NEG = -1e30


def _koleo_body(n, eps, xi_ref, xj_ref, out_ref, xj_bf_ref, b2_ref):
    i = pl.program_id(0)

    @pl.when(i == 0)
    def _():
        xj_all = xj_ref[...]
        xj_bf_ref[...] = xj_all.astype(jnp.bfloat16)
        ones = jnp.ones((1, xj_all.shape[1]), jnp.float32)
        b2_ref[...] = jax.lax.dot_general(
            ones, xj_all * xj_all, (((1,), (1,)), ((), ())),
            preferred_element_type=jnp.float32)
        out_ref[...] = jnp.zeros((1, 1), jnp.float32)

    xi_blk = xi_ref[...]
    xi_bf = (-2.0 * xi_blk).astype(jnp.bfloat16)

    s = jax.lax.dot_general(
        xi_bf, xj_bf_ref[...], (((1,), (1,)), ((), ())),
        preferred_element_type=jnp.float32)
    score = s + b2_ref[...]

    rows = i * _BM + jax.lax.broadcasted_iota(jnp.int32, (_BM, n), 0)
    cols = jax.lax.broadcasted_iota(jnp.int32, (_BM, n), 1)
    score = jnp.where(rows == cols, _NEG, score)

    m = jnp.max(score, axis=1, keepdims=True)
    a2 = jnp.sum(xi_blk * xi_blk, axis=1, keepdims=True)
    d2 = a2 + m
    lg = jnp.log(1.0 / (d2 * 0.5 + 1.0) ** 2 + eps)
    out_ref[...] += jnp.sum(lg, keepdims=True)


def kernel(xi, xj):
    eps = 1e-08
    n, k = xi.shape

    out = pl.pallas_call(
        functools.partial(_koleo_body, n, eps),
        grid=(n // _BM,),
        in_specs=[
            pl.BlockSpec((_BM, k), lambda i: (i, 0)),
            pl.BlockSpec((n, k), lambda i: (0, 0)),
        ],
        out_specs=pl.BlockSpec((1, 1), lambda i: (0, 0)),
        out_shape=jax.ShapeDtypeStruct((1, 1), jnp.float32),
        scratch_shapes=[
            pltpu.VMEM((n, k), jnp.bfloat16),
            pltpu.VMEM((1, n), jnp.float32),
        ],
        compiler_params=pltpu.CompilerParams(
            dimension_semantics=("arbitrary",)),
    )(xi, xj)
    return out[0, 0] / n


# speedup vs baseline: 15.2483x; 1.0377x over previous
import functools

import jax
import jax.numpy as jnp
from jax.experimental import pallas as pl
from jax.experimental.pallas import tpu as pltpu

_BM = 1024
_NEG = -1e30


def _koleo_body(n, eps, xi_ref, xj_ref, out_ref, xj_bf_ref, b2_ref):
    i = pl.program_id(0)

    @pl.when(i == 0)
    def _():
        xj_all = xj_ref[...]
        xj_bf_ref[...] = xj_all.astype(jnp.bfloat16)
        ones = jnp.ones((1, xj_all.shape[1]), jnp.float32)
        b2_ref[...] = jax.lax.dot_general(
            ones, xj_all * xj_all, (((1,), (1,)), ((), ())),
            preferred_element_type=jnp.float32)
        out_ref[...] = jnp.zeros((1, 1), jnp.float32)

    xi_blk = xi_ref[...]
    xi_bf = (-2.0 * xi_blk).astype(jnp.bfloat16)

    s = jax.lax.dot_general(
        xi_bf, xj_bf_ref[...], (((1,), (1,)), ((), ())),
        preferred_element_type=jnp.float32)
    score = s + b2_ref[...]

    rows = i * _BM + jax.lax.broadcasted_iota(jnp.int32, (_BM, n), 0)
    cols = jax.lax.broadcasted_iota(jnp.int32, (_BM, n), 1)
    score = jnp.where(rows == cols, _NEG, score)

    m = jnp.max(score, axis=1, keepdims=True)
    a2 = jnp.sum(xi_blk * xi_blk, axis=1, keepdims=True)
    d2 = a2 + m
    lg = jnp.log(1.0 / (d2 * 0.5 + 1.0) ** 2 + eps)
    out_ref[...] += jnp.sum(lg, keepdims=True)


def kernel(xi, xj):
    eps = 1e-08
    n, k = xi.shape

    out = pl.pallas_call(
        functools.partial(_koleo_body, n, eps),
        grid=(n // _BM,),
        in_specs=[
            pl.BlockSpec((_BM, k), lambda i: (i, 0)),
            pl.BlockSpec((n, k), lambda i: (0, 0)),
        ],
        out_specs=pl.BlockSpec((1, 1), lambda i: (0, 0)),
        out_shape=jax.ShapeDtypeStruct((1, 1), jnp.float32),
        scratch_shapes=[
            pltpu.VMEM((n, k), jnp.bfloat16),
            pltpu.VMEM((1, n), jnp.float32),
        ],
        compiler_params=pltpu.CompilerParams(
            dimension_semantics=("arbitrary",)),
    )(xi, xj)
    return out[0, 0] / n
